# baseline (speedup 1.0000x reference)
import jax
import jax.numpy as jnp
from jax import lax
from jax.experimental import pallas as pl
from jax.experimental.pallas import tpu as pltpu
from jax.experimental.pallas import tpu_sc as plsc

N = 10000
E = 320000
IN_DIM = 128
OUT_DIM = 32
H = 4
FEAT = 4

NC = 2
NS = 16
L = 16
HPC = H // NC
ZCOLS = HPC * OUT_DIM

STRIPE = 640
NPAD = STRIPE * NS
CHUNK = 128
BLK = 128
EB = 2 * BLK
EP = 327680
TPB = EP // EB // NS
NEG = -1e30
BN = 1000
BE = 2560


def _node_kernel(h_ref, w_ref, a12_ref, z2_ref, sd_ref):
    z = jnp.dot(h_ref[...], w_ref[...], preferred_element_type=jnp.float32)
    z2_ref[0] = z[:, :ZCOLS]
    z2_ref[1] = z[:, ZCOLS:]
    sd_ref[...] = jnp.dot(z, a12_ref[...], preferred_element_type=jnp.float32)


def _edge_kernel(ea_ref, g_ref, c_ref, fa_ref):
    fa = lax.dot_general(g_ref[...], ea_ref[...],
                         (((1,), (1,)), ((), ())),
                         preferred_element_type=jnp.float32) + c_ref[...]
    real = pl.program_id(0) < E // BE
    fa_ref[...] = jnp.where(real, fa, NEG)


def _norm_kernel(o_ref, s_ref, p_ref, out_ref):
    o = jnp.concatenate([o_ref[0], o_ref[1]], axis=1)
    denom = jnp.dot(s_ref[...], p_ref[...], preferred_element_type=jnp.float32)
    out_ref[...] = o / (denom + 1e-9)


class _PipeBufs:
    def __init__(self, refs):
        (self.srcb, self.dstb, self.fab0, self.fab1,
         self.gidxA, self.gidxB, self.dstcA, self.dstcB,
         self.ex0A, self.ex0B, self.ex1A, self.ex1B, self.zrowsA, self.zrowsB,
         self.sem_idx, self.sem_g, self.sem_sc) = refs


def _sc_body(z2, sdt, fat, srcl, dstl, zero2, zero1,
             out_hbm, s_hbm,
             sa0, sa1, da0, da1, *rest):
    P0 = _PipeBufs(rest[0:17])
    P1 = _PipeBufs(rest[17:34])
    out_acc, sacc0, sacc1 = rest[34:37]
    c = lax.axis_index("core")
    sid = lax.axis_index("subcore")
    h0 = 2 * c

    pltpu.sync_copy(sdt.at[pl.ds(h0 * N, N)], sa0)
    pltpu.sync_copy(sdt.at[pl.ds((h0 + 1) * N, N)], sa1)
    pltpu.sync_copy(sdt.at[pl.ds((H + h0) * N, N)], da0)
    pltpu.sync_copy(sdt.at[pl.ds((H + h0 + 1) * N, N)], da1)

    base = sid * STRIPE
    pltpu.sync_copy(zero2, P0.zrowsA)
    pltpu.sync_copy(zero1, P0.ex0A)

    @pl.loop(0, STRIPE // CHUNK)
    def _(k):
        ds = pl.ds(base + k * CHUNK, CHUNK)
        pltpu.sync_copy(P0.zrowsA, out_acc.at[ds])
        pltpu.sync_copy(P0.ex0A, sacc0.at[ds])
        pltpu.sync_copy(P0.ex0A, sacc1.at[ds])

    plsc.subcore_barrier()

    def issue_idx(t, P):
        off = (t * NS + sid) * EB
        pltpu.async_copy(srcl.at[pl.ds(off, EB)], P.srcb, P.sem_idx)
        pltpu.async_copy(dstl.at[pl.ds(off, EB)], P.dstb, P.sem_idx)
        pltpu.async_copy(fat.at[pl.ds(h0 * EP + off, EB)], P.fab0, P.sem_idx)
        pltpu.async_copy(fat.at[pl.ds((h0 + 1) * EP + off, EB)], P.fab1,
                         P.sem_idx)

    def wait_idx(P):
        pltpu.make_async_copy(srcl.at[pl.ds(0, EB)], P.srcb, P.sem_idx).wait()
        pltpu.make_async_copy(dstl.at[pl.ds(0, EB)], P.dstb, P.sem_idx).wait()
        pltpu.make_async_copy(fat.at[pl.ds(0, EB)], P.fab0, P.sem_idx).wait()
        pltpu.make_async_copy(fat.at[pl.ds(0, EB)], P.fab1, P.sem_idx).wait()

    def wait_scatter(P):
        pltpu.make_async_copy(P.zrowsA, out_acc.at[P.dstcA], P.sem_sc).wait()
        pltpu.make_async_copy(P.zrowsB, out_acc.at[P.dstcB], P.sem_sc).wait()
        pltpu.make_async_copy(P.ex0A, sacc0.at[P.dstcA], P.sem_sc).wait()
        pltpu.make_async_copy(P.ex0B, sacc0.at[P.dstcB], P.sem_sc).wait()
        pltpu.make_async_copy(P.ex1A, sacc1.at[P.dstcA], P.sem_sc).wait()
        pltpu.make_async_copy(P.ex1B, sacc1.at[P.dstcB], P.sem_sc).wait()

    def ex_groups(P, half):
        gidx, dstc, ex0, ex1 = ((P.gidxA, P.dstcA, P.ex0A, P.ex1A),
                                (P.gidxB, P.dstcB, P.ex0B, P.ex1B))[half]
        for gg in range(BLK // L):
            g = half * (BLK // L) + gg
            sl = pl.ds(g * L, L)
            hsl = pl.ds(gg * L, L)
            s16 = P.srcb[sl]
            d16 = P.dstb[sl]
            gidx[hsl] = s16 + c * N
            dstc[hsl] = d16
            for saR, daR, faR, exR in ((sa0, da0, P.fab0, ex0),
                                       (sa1, da1, P.fab1, ex1)):
                x = (plsc.load_gather(saR, [s16])
                     + plsc.load_gather(daR, [d16]) + faR[sl])
                x = jnp.maximum(x, x * 0.2)
                exR[hsl] = jnp.exp(x)

    def phase_a(i, t, P):
        @pl.when(i > 0)
        def _():
            wait_scatter(P)

        wait_idx(P)
        ex_groups(P, 0)
        ghA = pltpu.async_copy(z2.at[P.gidxA], P.zrowsA, P.sem_g)
        ex_groups(P, 1)
        ghB = pltpu.async_copy(z2.at[P.gidxB], P.zrowsB, P.sem_g)

        @pl.when(i < TPB // 2 - 1)
        def _():
            issue_idx(t + 2, P)

        return ghA, ghB

    def phase_b(P, gh):
        gh[0].wait()
        gh[1].wait()

        dn = lax.GatherDimensionNumbers(offset_dims=(),
                                        collapsed_slice_dims=(0,),
                                        start_index_map=(0,))

        @pl.loop(0, BLK // L)
        def _(g):
            for zr, e0, e1 in ((P.zrowsA, P.ex0A, P.ex1A),
                               (P.zrowsB, P.ex0B, P.ex1B)):
                w0v = e0[pl.ds(g * L, L)]
                w1v = e1[pl.ds(g * L, L)]
                for el in range(L):
                    lane = jnp.full((L, 1), el, jnp.int32)
                    w0 = lax.gather(w0v, lane, dn, slice_sizes=(1,),
                                    mode=lax.GatherScatterMode.PROMISE_IN_BOUNDS)
                    w1 = lax.gather(w1v, lane, dn, slice_sizes=(1,),
                                    mode=lax.GatherScatterMode.PROMISE_IN_BOUNDS)
                    e = g * L + el
                    for cg in range(ZCOLS // L):
                        w = w0 if cg < OUT_DIM // L else w1
                        csl = pl.ds(cg * L, L)
                        zr[e, csl] = zr[e, csl] * w

        pltpu.async_copy(P.zrowsA, out_acc.at[P.dstcA], P.sem_sc, add=True)
        pltpu.async_copy(P.zrowsB, out_acc.at[P.dstcB], P.sem_sc, add=True)
        pltpu.async_copy(P.ex0A, sacc0.at[P.dstcA], P.sem_sc, add=True)
        pltpu.async_copy(P.ex0B, sacc0.at[P.dstcB], P.sem_sc, add=True)
        pltpu.async_copy(P.ex1A, sacc1.at[P.dstcA], P.sem_sc, add=True)
        pltpu.async_copy(P.ex1B, sacc1.at[P.dstcB], P.sem_sc, add=True)

    issue_idx(0, P0)
    issue_idx(1, P1)

    @pl.loop(0, TPB // 2)
    def _(i):
        gh0 = phase_a(i, 2 * i, P0)
        gh1 = phase_a(i, 2 * i + 1, P1)
        phase_b(P0, gh0)
        phase_b(P1, gh1)

    wait_scatter(P0)
    wait_scatter(P1)
    plsc.subcore_barrier()

    @pl.loop(0, STRIPE // CHUNK)
    def _(k):
        ds = pl.ds(base + k * CHUNK, CHUNK)
        pltpu.sync_copy(out_acc.at[ds], P0.zrowsA)
        pltpu.sync_copy(P0.zrowsA, out_hbm.at[pl.ds(c * NPAD + base
                                                    + k * CHUNK, CHUNK)])
        pltpu.sync_copy(sacc0.at[ds], P0.ex0A)
        pltpu.sync_copy(P0.ex0A, s_hbm.at[pl.ds(h0 * NPAD + base + k * CHUNK,
                                                CHUNK)])
        pltpu.sync_copy(sacc1.at[ds], P0.ex1A)
        pltpu.sync_copy(P0.ex1A, s_hbm.at[pl.ds((h0 + 1) * NPAD + base
                                                + k * CHUNK, CHUNK)])


def kernel(h, edge_index, edge_attr, W, Wf, bf, a):
    f32 = jnp.float32
    src = edge_index[0]
    dst = edge_index[1]

    w_all = W.transpose(1, 0, 2).reshape(IN_DIM, H * OUT_DIM)
    a1 = a[:, :OUT_DIM]
    a2 = a[:, OUT_DIM:2 * OUT_DIM]
    a3 = a[:, 2 * OUT_DIM:]
    eye = jnp.eye(H, dtype=f32)
    a12 = jnp.concatenate(
        [jnp.einsum("ho,hk->hok", a1, eye).reshape(H * OUT_DIM, H),
         jnp.einsum("ho,hk->hok", a2, eye).reshape(H * OUT_DIM, H)], axis=1)
    g8 = jnp.zeros((8, FEAT), f32).at[:H].set(
        jnp.einsum("hfo,ho->hf", Wf, a3))
    c8 = jnp.zeros((8, 1), f32).at[:H, 0].set(jnp.einsum("ho,ho->h", bf, a3))

    z2, sd = pl.pallas_call(
        _node_kernel,
        grid=(N // BN,),
        in_specs=[pl.BlockSpec((BN, IN_DIM), lambda i: (i, 0)),
                  pl.BlockSpec((IN_DIM, H * OUT_DIM), lambda i: (0, 0)),
                  pl.BlockSpec((IN_DIM, 2 * H), lambda i: (0, 0))],
        out_specs=[pl.BlockSpec((NC, BN, ZCOLS), lambda i: (0, i, 0)),
                   pl.BlockSpec((BN, 2 * H), lambda i: (i, 0))],
        out_shape=[jax.ShapeDtypeStruct((NC, N, ZCOLS), f32),
                   jax.ShapeDtypeStruct((N, 2 * H), f32)],
    )(h, w_all, a12)

    fat = pl.pallas_call(
        _edge_kernel,
        grid=(EP // BE,),
        in_specs=[pl.BlockSpec((BE, FEAT),
                               lambda i: (jnp.minimum(i, E // BE - 1), 0)),
                  pl.BlockSpec((8, FEAT), lambda i: (0, 0)),
                  pl.BlockSpec((8, 1), lambda i: (0, 0))],
        out_specs=pl.BlockSpec((8, BE), lambda i: (0, i)),
        out_shape=jax.ShapeDtypeStruct((8, EP), f32),
    )(edge_attr, g8, c8)

    zflat = z2.reshape(NC * N, ZCOLS)
    sdt = sd.T.reshape(2 * H * N)
    src_p = jnp.concatenate([src, jnp.zeros(EP - E, jnp.int32)])
    dst_p = jnp.concatenate([dst, jnp.zeros(EP - E, jnp.int32)])
    fat1 = fat.reshape(8 * EP)
    zero2 = jnp.zeros((CHUNK, ZCOLS), f32)
    zero1 = jnp.zeros((CHUNK,), f32)

    mesh = plsc.VectorSubcoreMesh(core_axis_name="core",
                                  subcore_axis_name="subcore")
    pipe_bufs = [pltpu.VMEM((EB,), jnp.int32),
                 pltpu.VMEM((EB,), jnp.int32),
                 pltpu.VMEM((EB,), f32),
                 pltpu.VMEM((EB,), f32),
                 pltpu.VMEM((BLK,), jnp.int32),
                 pltpu.VMEM((BLK,), jnp.int32),
                 pltpu.VMEM((BLK,), jnp.int32),
                 pltpu.VMEM((BLK,), jnp.int32),
                 pltpu.VMEM((BLK,), f32),
                 pltpu.VMEM((BLK,), f32),
                 pltpu.VMEM((BLK,), f32),
                 pltpu.VMEM((BLK,), f32),
                 pltpu.VMEM((BLK, ZCOLS), f32),
                 pltpu.VMEM((BLK, ZCOLS), f32),
                 pltpu.SemaphoreType.DMA,
                 pltpu.SemaphoreType.DMA,
                 pltpu.SemaphoreType.DMA]
    sc_call = pl.kernel(
        _sc_body,
        compiler_params=pltpu.CompilerParams(needs_layout_passes=False,
                                             use_tc_tiling_on_sc=False),
        out_type=[jax.ShapeDtypeStruct((NC * NPAD, ZCOLS), f32),
                  jax.ShapeDtypeStruct((H * NPAD,), f32)],
        mesh=mesh,
        scratch_types=[pltpu.VMEM((N,), f32),
                       pltpu.VMEM((N,), f32),
                       pltpu.VMEM((N,), f32),
                       pltpu.VMEM((N,), f32)]
        + pipe_bufs + pipe_bufs
        + [pltpu.VMEM_SHARED((NPAD, ZCOLS), f32),
           pltpu.VMEM_SHARED((NPAD,), f32),
           pltpu.VMEM_SHARED((NPAD,), f32)],
    )
    out_flat, s_flat = sc_call(zflat, sdt, fat1, src_p, dst_p, zero2, zero1)

    o2 = out_flat.reshape(NC, NPAD, ZCOLS)
    s8 = jnp.concatenate(
        [s_flat.reshape(H, NPAD)[:, :N].T, jnp.zeros((N, 8 - H), f32)], axis=1)
    p8 = jnp.zeros((8, H * OUT_DIM), f32).at[
        jnp.arange(H).repeat(OUT_DIM), jnp.arange(H * OUT_DIM)].set(1.0)
    out = pl.pallas_call(
        _norm_kernel,
        grid=(N // BN,),
        in_specs=[pl.BlockSpec((NC, BN, ZCOLS), lambda i: (0, i, 0)),
                  pl.BlockSpec((BN, 8), lambda i: (i, 0)),
                  pl.BlockSpec((8, H * OUT_DIM), lambda i: (0, 0))],
        out_specs=pl.BlockSpec((BN, H * OUT_DIM), lambda i: (i, 0)),
        out_shape=jax.ShapeDtypeStruct((N, H * OUT_DIM), f32),
    )(o2, s8, p8)
    return out


# speedup vs baseline: 1.3104x; 1.3104x over previous
import jax
import jax.numpy as jnp
from jax import lax
from jax.experimental import pallas as pl
from jax.experimental.pallas import tpu as pltpu
from jax.experimental.pallas import tpu_sc as plsc

N = 10000
E = 320000
IN_DIM = 128
OUT_DIM = 32
H = 4
FEAT = 4

NC = 2
NS = 16
L = 16
HPC = H // NC
ZCOLS = HPC * OUT_DIM

STRIPE = 640
NPAD = STRIPE * NS
CHUNK = 128
BLK = 128
EB = 2 * BLK
EP = 327680
TPB = EP // EB // NS
NEG = -1e30
BN = 1000
BE = 6400


def _node_kernel(h_ref, w_ref, a12_ref, z2_ref, sd_ref):
    z = jnp.dot(h_ref[...], w_ref[...], preferred_element_type=jnp.float32)
    z2_ref[0] = z[:, :ZCOLS]
    z2_ref[1] = z[:, ZCOLS:]
    sd_ref[...] = jnp.dot(z, a12_ref[...], preferred_element_type=jnp.float32)


def _edge_kernel(ea_ref, g_ref, c_ref, fa_ref):
    fa_ref[...] = (
        jnp.dot(g_ref[...], ea_ref[...], preferred_element_type=jnp.float32)
        + c_ref[...]
    )


def _norm_kernel(o_ref, s_ref, p_ref, out_ref):
    o = jnp.concatenate([o_ref[0], o_ref[1]], axis=1)
    denom = jnp.dot(s_ref[...], p_ref[...], preferred_element_type=jnp.float32)
    out_ref[...] = o / (denom + 1e-9)


class _PipeBufs:
    def __init__(self, refs):
        (self.srcb, self.dstb, self.fab0, self.fab1,
         self.gidxA, self.gidxB, self.dstcA, self.dstcB,
         self.ex0A, self.ex0B, self.ex1A, self.ex1B, self.zrowsA, self.zrowsB,
         self.sem_idx, self.sem_g, self.sem_sc) = refs


def _sc_body(z2, sdt, fat, srcl, dstl, zero2, zero1,
             out_hbm, s_hbm,
             sa0, sa1, da0, da1, *rest):
    P0 = _PipeBufs(rest[0:17])
    P1 = _PipeBufs(rest[17:34])
    out_acc, sacc0, sacc1 = rest[34:37]
    c = lax.axis_index("core")
    sid = lax.axis_index("subcore")
    h0 = 2 * c

    pltpu.sync_copy(sdt.at[pl.ds(h0 * N, N)], sa0)
    pltpu.sync_copy(sdt.at[pl.ds((h0 + 1) * N, N)], sa1)
    pltpu.sync_copy(sdt.at[pl.ds((H + h0) * N, N)], da0)
    pltpu.sync_copy(sdt.at[pl.ds((H + h0 + 1) * N, N)], da1)

    base = sid * STRIPE
    pltpu.sync_copy(zero2, P0.zrowsA)
    pltpu.sync_copy(zero1, P0.ex0A)

    @pl.loop(0, STRIPE // CHUNK)
    def _(k):
        ds = pl.ds(base + k * CHUNK, CHUNK)
        pltpu.sync_copy(P0.zrowsA, out_acc.at[ds])
        pltpu.sync_copy(P0.ex0A, sacc0.at[ds])
        pltpu.sync_copy(P0.ex0A, sacc1.at[ds])

    plsc.subcore_barrier()

    def issue_idx(t, P):
        off = (t * NS + sid) * EB
        pltpu.async_copy(srcl.at[pl.ds(off, EB)], P.srcb, P.sem_idx)
        pltpu.async_copy(dstl.at[pl.ds(off, EB)], P.dstb, P.sem_idx)
        pltpu.async_copy(fat.at[pl.ds(h0 * EP + off, EB)], P.fab0, P.sem_idx)
        pltpu.async_copy(fat.at[pl.ds((h0 + 1) * EP + off, EB)], P.fab1,
                         P.sem_idx)

    def wait_idx(P):
        pltpu.make_async_copy(srcl.at[pl.ds(0, EB)], P.srcb, P.sem_idx).wait()
        pltpu.make_async_copy(dstl.at[pl.ds(0, EB)], P.dstb, P.sem_idx).wait()
        pltpu.make_async_copy(fat.at[pl.ds(0, EB)], P.fab0, P.sem_idx).wait()
        pltpu.make_async_copy(fat.at[pl.ds(0, EB)], P.fab1, P.sem_idx).wait()

    def wait_scatter(P):
        pltpu.make_async_copy(P.zrowsA, out_acc.at[P.dstcA], P.sem_sc).wait()
        pltpu.make_async_copy(P.zrowsB, out_acc.at[P.dstcB], P.sem_sc).wait()
        pltpu.make_async_copy(P.ex0A, sacc0.at[P.dstcA], P.sem_sc).wait()
        pltpu.make_async_copy(P.ex0B, sacc0.at[P.dstcB], P.sem_sc).wait()
        pltpu.make_async_copy(P.ex1A, sacc1.at[P.dstcA], P.sem_sc).wait()
        pltpu.make_async_copy(P.ex1B, sacc1.at[P.dstcB], P.sem_sc).wait()

    def ex_groups(P, half):
        gidx, dstc, ex0, ex1 = ((P.gidxA, P.dstcA, P.ex0A, P.ex1A),
                                (P.gidxB, P.dstcB, P.ex0B, P.ex1B))[half]
        for gg in range(BLK // L):
            g = half * (BLK // L) + gg
            sl = pl.ds(g * L, L)
            hsl = pl.ds(gg * L, L)
            s16 = P.srcb[sl]
            d16 = P.dstb[sl]
            gidx[hsl] = s16 + c * N
            dstc[hsl] = d16
            for saR, daR, faR, exR in ((sa0, da0, P.fab0, ex0),
                                       (sa1, da1, P.fab1, ex1)):
                x = (plsc.load_gather(saR, [s16])
                     + plsc.load_gather(daR, [d16]) + faR[sl])
                x = jnp.maximum(x, x * 0.2)
                exR[hsl] = jnp.exp(x)

    def phase_a(i, t, P):
        @pl.when(i > 0)
        def _():
            wait_scatter(P)

        wait_idx(P)
        ex_groups(P, 0)
        ghA = pltpu.async_copy(z2.at[P.gidxA], P.zrowsA, P.sem_g)
        ex_groups(P, 1)
        ghB = pltpu.async_copy(z2.at[P.gidxB], P.zrowsB, P.sem_g)

        @pl.when(i < TPB // 2 - 1)
        def _():
            issue_idx(t + 2, P)

        return ghA, ghB

    def phase_b(P, gh):
        gh[0].wait()
        gh[1].wait()

        dn = lax.GatherDimensionNumbers(offset_dims=(),
                                        collapsed_slice_dims=(0,),
                                        start_index_map=(0,))

        @pl.loop(0, BLK // L)
        def _(g):
            for zr, e0, e1 in ((P.zrowsA, P.ex0A, P.ex1A),
                               (P.zrowsB, P.ex0B, P.ex1B)):
                w0v = e0[pl.ds(g * L, L)]
                w1v = e1[pl.ds(g * L, L)]
                for el in range(L):
                    lane = jnp.full((L, 1), el, jnp.int32)
                    w0 = lax.gather(w0v, lane, dn, slice_sizes=(1,),
                                    mode=lax.GatherScatterMode.PROMISE_IN_BOUNDS)
                    w1 = lax.gather(w1v, lane, dn, slice_sizes=(1,),
                                    mode=lax.GatherScatterMode.PROMISE_IN_BOUNDS)
                    e = g * L + el
                    for cg in range(ZCOLS // L):
                        w = w0 if cg < OUT_DIM // L else w1
                        csl = pl.ds(cg * L, L)
                        zr[e, csl] = zr[e, csl] * w

        pltpu.async_copy(P.zrowsA, out_acc.at[P.dstcA], P.sem_sc, add=True)
        pltpu.async_copy(P.zrowsB, out_acc.at[P.dstcB], P.sem_sc, add=True)
        pltpu.async_copy(P.ex0A, sacc0.at[P.dstcA], P.sem_sc, add=True)
        pltpu.async_copy(P.ex0B, sacc0.at[P.dstcB], P.sem_sc, add=True)
        pltpu.async_copy(P.ex1A, sacc1.at[P.dstcA], P.sem_sc, add=True)
        pltpu.async_copy(P.ex1B, sacc1.at[P.dstcB], P.sem_sc, add=True)

    issue_idx(0, P0)
    issue_idx(1, P1)

    @pl.loop(0, TPB // 2)
    def _(i):
        gh0 = phase_a(i, 2 * i, P0)
        gh1 = phase_a(i, 2 * i + 1, P1)
        phase_b(P0, gh0)
        phase_b(P1, gh1)

    wait_scatter(P0)
    wait_scatter(P1)
    plsc.subcore_barrier()

    @pl.loop(0, STRIPE // CHUNK)
    def _(k):
        ds = pl.ds(base + k * CHUNK, CHUNK)
        pltpu.sync_copy(out_acc.at[ds], P0.zrowsA)
        pltpu.sync_copy(P0.zrowsA, out_hbm.at[pl.ds(c * NPAD + base
                                                    + k * CHUNK, CHUNK)])
        pltpu.sync_copy(sacc0.at[ds], P0.ex0A)
        pltpu.sync_copy(P0.ex0A, s_hbm.at[pl.ds(h0 * NPAD + base + k * CHUNK,
                                                CHUNK)])
        pltpu.sync_copy(sacc1.at[ds], P0.ex1A)
        pltpu.sync_copy(P0.ex1A, s_hbm.at[pl.ds((h0 + 1) * NPAD + base
                                                + k * CHUNK, CHUNK)])


def kernel(h, edge_index, edge_attr, W, Wf, bf, a):
    f32 = jnp.float32
    src = edge_index[0]
    dst = edge_index[1]

    w_all = W.transpose(1, 0, 2).reshape(IN_DIM, H * OUT_DIM)
    a1 = a[:, :OUT_DIM]
    a2 = a[:, OUT_DIM:2 * OUT_DIM]
    a3 = a[:, 2 * OUT_DIM:]
    eye = jnp.eye(H, dtype=f32)
    a12 = jnp.concatenate(
        [jnp.einsum("ho,hk->hok", a1, eye).reshape(H * OUT_DIM, H),
         jnp.einsum("ho,hk->hok", a2, eye).reshape(H * OUT_DIM, H)], axis=1)
    g8 = jnp.zeros((8, 8), f32).at[:H, :FEAT].set(
        jnp.einsum("hfo,ho->hf", Wf, a3))
    c8 = jnp.zeros((8, 1), f32).at[:H, 0].set(jnp.einsum("ho,ho->h", bf, a3))
    ea8 = jnp.concatenate([edge_attr.T, jnp.zeros((8 - FEAT, E), f32)], axis=0)

    z2, sd = pl.pallas_call(
        _node_kernel,
        grid=(N // BN,),
        in_specs=[pl.BlockSpec((BN, IN_DIM), lambda i: (i, 0)),
                  pl.BlockSpec((IN_DIM, H * OUT_DIM), lambda i: (0, 0)),
                  pl.BlockSpec((IN_DIM, 2 * H), lambda i: (0, 0))],
        out_specs=[pl.BlockSpec((NC, BN, ZCOLS), lambda i: (0, i, 0)),
                   pl.BlockSpec((BN, 2 * H), lambda i: (i, 0))],
        out_shape=[jax.ShapeDtypeStruct((NC, N, ZCOLS), f32),
                   jax.ShapeDtypeStruct((N, 2 * H), f32)],
    )(h, w_all, a12)

    fat = pl.pallas_call(
        _edge_kernel,
        grid=(E // BE,),
        in_specs=[pl.BlockSpec((8, BE), lambda i: (0, i)),
                  pl.BlockSpec((8, 8), lambda i: (0, 0)),
                  pl.BlockSpec((8, 1), lambda i: (0, 0))],
        out_specs=pl.BlockSpec((8, BE), lambda i: (0, i)),
        out_shape=jax.ShapeDtypeStruct((8, E), f32),
    )(ea8, g8, c8)

    zflat = z2.reshape(NC * N, ZCOLS)
    sdt = sd.T.reshape(2 * H * N)
    src_p = jnp.concatenate([src, jnp.zeros(EP - E, jnp.int32)])
    dst_p = jnp.concatenate([dst, jnp.zeros(EP - E, jnp.int32)])
    fat1 = jnp.pad(fat, ((0, 0), (0, EP - E)),
                   constant_values=NEG).reshape(8 * EP)
    zero2 = jnp.zeros((CHUNK, ZCOLS), f32)
    zero1 = jnp.zeros((CHUNK,), f32)

    mesh = plsc.VectorSubcoreMesh(core_axis_name="core",
                                  subcore_axis_name="subcore")
    pipe_bufs = [pltpu.VMEM((EB,), jnp.int32),
                 pltpu.VMEM((EB,), jnp.int32),
                 pltpu.VMEM((EB,), f32),
                 pltpu.VMEM((EB,), f32),
                 pltpu.VMEM((BLK,), jnp.int32),
                 pltpu.VMEM((BLK,), jnp.int32),
                 pltpu.VMEM((BLK,), jnp.int32),
                 pltpu.VMEM((BLK,), jnp.int32),
                 pltpu.VMEM((BLK,), f32),
                 pltpu.VMEM((BLK,), f32),
                 pltpu.VMEM((BLK,), f32),
                 pltpu.VMEM((BLK,), f32),
                 pltpu.VMEM((BLK, ZCOLS), f32),
                 pltpu.VMEM((BLK, ZCOLS), f32),
                 pltpu.SemaphoreType.DMA,
                 pltpu.SemaphoreType.DMA,
                 pltpu.SemaphoreType.DMA]
    sc_call = pl.kernel(
        _sc_body,
        compiler_params=pltpu.CompilerParams(needs_layout_passes=False,
                                             use_tc_tiling_on_sc=False),
        out_type=[jax.ShapeDtypeStruct((NC * NPAD, ZCOLS), f32),
                  jax.ShapeDtypeStruct((H * NPAD,), f32)],
        mesh=mesh,
        scratch_types=[pltpu.VMEM((N,), f32),
                       pltpu.VMEM((N,), f32),
                       pltpu.VMEM((N,), f32),
                       pltpu.VMEM((N,), f32)]
        + pipe_bufs + pipe_bufs
        + [pltpu.VMEM_SHARED((NPAD, ZCOLS), f32),
           pltpu.VMEM_SHARED((NPAD,), f32),
           pltpu.VMEM_SHARED((NPAD,), f32)],
    )
    out_flat, s_flat = sc_call(zflat, sdt, fat1, src_p, dst_p, zero2, zero1)

    o2 = out_flat.reshape(NC, NPAD, ZCOLS)
    s8 = jnp.concatenate(
        [s_flat.reshape(H, NPAD)[:, :N].T, jnp.zeros((N, 8 - H), f32)], axis=1)
    p8 = jnp.zeros((8, H * OUT_DIM), f32).at[
        jnp.arange(H).repeat(OUT_DIM), jnp.arange(H * OUT_DIM)].set(1.0)
    out = pl.pallas_call(
        _norm_kernel,
        grid=(N // BN,),
        in_specs=[pl.BlockSpec((NC, BN, ZCOLS), lambda i: (0, i, 0)),
                  pl.BlockSpec((BN, 8), lambda i: (i, 0)),
                  pl.BlockSpec((8, H * OUT_DIM), lambda i: (0, 0))],
        out_specs=pl.BlockSpec((BN, H * OUT_DIM), lambda i: (i, 0)),
        out_shape=jax.ShapeDtypeStruct((N, H * OUT_DIM), f32),
    )(o2, s8, p8)
    return out
